# baseline (speedup 1.0000x reference)
import functools

import numpy as np
import jax
import jax.numpy as jnp
from jax import lax
from jax.experimental import pallas as pl
from jax.experimental.pallas import tpu as pltpu
from jax.experimental.pallas import tpu_sc as plsc

N = 8192
INPUT_DIM = 4
HID = 256
OUT = 4
NF = 10
NL = 8
NC = 8
B = 256
NBLK = N // B + NC
NROWS = NBLK * B
PADW = 128
CHUNK = 80
NCHUNK = NROWS // CHUNK
NCC, NSC = 2, 16
NW = NCC * NSC
CPW = NCHUNK // NW

_PERM_SIN = np.array([4 + i * 8 + 2 * j for i in range(NF) for j in range(INPUT_DIM)])
_PERM_COS = _PERM_SIN + 1
_FMAT = np.zeros((INPUT_DIM, NF * INPUT_DIM), np.float32)
for _i in range(NF):
    for _j in range(INPUT_DIM):
        _FMAT[_j, _i * INPUT_DIM + _j] = 2.0 ** _i * np.pi


def _mlp_block(be_ref, tok_ref, fm_ref, w0a_ref, w0s_ref, w0c_ref, b0_ref,
               wb_ref, bb_ref, sc_ref, wo_ref, bo_ref, out_ref):
    t = tok_ref[...]
    x4 = t[:, 0:4]
    xn = jnp.where(lax.broadcasted_iota(jnp.int32, (B, INPUT_DIM), 1) < 3,
                   x4 / t[:, 3:4], x4)
    ang = lax.dot_general(xn, fm_ref[...], (((1,), (0,)), ((), ())),
                          precision=lax.Precision.HIGHEST,
                          preferred_element_type=jnp.float32)
    sn = jnp.sin(ang)
    cs = jnp.cos(ang)
    h = (jnp.dot(xn, w0a_ref[0], preferred_element_type=jnp.float32)
         + jnp.dot(sn, w0s_ref[0], preferred_element_type=jnp.float32)
         + jnp.dot(cs, w0c_ref[0], preferred_element_type=jnp.float32)
         + b0_ref[0])
    h = jnp.maximum(h, 0.0)
    for l in range(NL - 1):
        z = jnp.dot(h, wb_ref[0, l], preferred_element_type=jnp.float32) + bb_ref[0, l]
        h = sc_ref[0, l, 0] * jnp.maximum(z, 0.0) + h
    o = jnp.dot(h, wo_ref[0], preferred_element_type=jnp.float32) + bo_ref[0]
    out_ref[...] = o / t[:, 4:5]


@functools.cache
def _sc_kernels():
    mesh = plsc.VectorSubcoreMesh(core_axis_name="c", subcore_axis_name="s")

    @functools.partial(
        pl.kernel, mesh=mesh,
        out_type=jax.ShapeDtypeStruct((NROWS, PADW), jnp.float32),
        scratch_types=[pltpu.VMEM((CHUNK,), jnp.int32),
                       pltpu.VMEM((CHUNK, PADW), jnp.float32),
                       pltpu.SemaphoreType.DMA])
    def sc_gather(table, idx2, out, idx_v, rows_v, sem):
        wid = lax.axis_index("s") * NCC + lax.axis_index("c")
        for k in range(CPW):
            cid = wid * CPW + k
            pltpu.sync_copy(idx2.at[cid], idx_v)
            pltpu.async_copy(table.at[idx_v], rows_v, sem).wait()
            pltpu.sync_copy(rows_v, out.at[pl.ds(cid * CHUNK, CHUNK)])

    @functools.partial(
        pl.kernel, mesh=mesh,
        out_type=jax.ShapeDtypeStruct((N + 8, PADW), jnp.float32),
        scratch_types=[pltpu.VMEM((CHUNK,), jnp.int32),
                       pltpu.VMEM((CHUNK, PADW), jnp.float32),
                       pltpu.SemaphoreType.DMA])
    def sc_scatter(vals, idx2, out, idx_v, rows_v, sem):
        wid = lax.axis_index("s") * NCC + lax.axis_index("c")
        for k in range(CPW):
            cid = wid * CPW + k
            pltpu.sync_copy(idx2.at[cid], idx_v)
            pltpu.sync_copy(vals.at[pl.ds(cid * CHUNK, CHUNK)], rows_v)
            pltpu.async_copy(rows_v, out.at[idx_v], sem).wait()

    return sc_gather, sc_scatter


def _routing(layer_id):
    e = layer_id.astype(jnp.int32)
    onehot = (e[:, None] == jnp.arange(NC, dtype=jnp.int32)).astype(jnp.int32)
    rank = jnp.take_along_axis(jnp.cumsum(onehot, axis=0) - onehot,
                               e[:, None], axis=1)[:, 0]
    counts = jnp.sum(onehot, axis=0)
    padded = ((counts + B - 1) // B) * B
    starts = jnp.concatenate(
        [jnp.zeros((1,), jnp.int32), jnp.cumsum(padded)[:-1].astype(jnp.int32)])
    dest = jnp.take(starts, e) + rank
    gidx = jnp.zeros((NROWS,), jnp.int32).at[dest].set(
        jnp.arange(N, dtype=jnp.int32))
    valid = jnp.zeros((NROWS,), jnp.bool_).at[dest].set(True)
    sidx = jnp.where(valid, gidx, N)
    block_expert = (jnp.searchsorted(
        starts, jnp.arange(NBLK, dtype=jnp.int32) * B, side="right") - 1
    ).astype(jnp.int32)
    return gidx, sidx, block_expert


def kernel(x, layer_id, input_dim, W0, b0, Wb, bb, scalars, Wo, bo):
    gidx, sidx, block_expert = _routing(layer_id)

    xpad = jnp.concatenate(
        [x, input_dim[:, None], jnp.zeros((N, PADW - INPUT_DIM - 1), jnp.float32)],
        axis=1)

    sc_gather, sc_scatter = _sc_kernels()
    tok = sc_gather(xpad, gidx.reshape(NCHUNK, CHUNK))

    fmat = jnp.asarray(_FMAT)
    w0a = W0[:, :INPUT_DIM, :]
    w0s = W0[:, _PERM_SIN, :]
    w0c = W0[:, _PERM_COS, :]
    b0r = b0[:, None, :]
    scl3 = scalars[:, :, None]
    wo16 = jnp.zeros((NC, HID, PADW), jnp.float32).at[:, :, :OUT].set(Wo)
    bo16 = jnp.zeros((NC, 1, PADW), jnp.float32).at[:, 0, :OUT].set(bo)

    grid_spec = pltpu.PrefetchScalarGridSpec(
        num_scalar_prefetch=1,
        grid=(NBLK,),
        in_specs=[
            pl.BlockSpec((B, PADW), lambda i, be: (i, 0)),
            pl.BlockSpec((INPUT_DIM, NF * INPUT_DIM), lambda i, be: (0, 0)),
            pl.BlockSpec((1, INPUT_DIM, HID), lambda i, be: (be[i], 0, 0)),
            pl.BlockSpec((1, NF * INPUT_DIM, HID), lambda i, be: (be[i], 0, 0)),
            pl.BlockSpec((1, NF * INPUT_DIM, HID), lambda i, be: (be[i], 0, 0)),
            pl.BlockSpec((1, 1, HID), lambda i, be: (be[i], 0, 0)),
            pl.BlockSpec((1, NL - 1, HID, HID), lambda i, be: (be[i], 0, 0, 0)),
            pl.BlockSpec((1, NL - 1, HID), lambda i, be: (be[i], 0, 0)),
            pl.BlockSpec((1, NL - 1, 1), lambda i, be: (be[i], 0, 0)),
            pl.BlockSpec((1, HID, PADW), lambda i, be: (be[i], 0, 0)),
            pl.BlockSpec((1, 1, PADW), lambda i, be: (be[i], 0, 0)),
        ],
        out_specs=pl.BlockSpec((B, PADW), lambda i, be: (i, 0)),
    )
    vals = pl.pallas_call(
        _mlp_block,
        grid_spec=grid_spec,
        out_shape=jax.ShapeDtypeStruct((NROWS, PADW), jnp.float32),
    )(block_expert, tok, fmat, w0a, w0s, w0c, b0r, Wb, bb, scl3, wo16, bo16)

    scat = sc_scatter(vals, sidx.reshape(NCHUNK, CHUNK))
    return scat[:N, :OUT]
